# Initial kernel scaffold; baseline (speedup 1.0000x reference)
import jax
import jax.numpy as jnp
from jax.experimental import pallas as pl


def kernel(attn_weights):
    raise NotImplementedError("write your pallas kernel here")


# speedup vs baseline: 1.2141x; 1.2141x over previous
import functools

import jax
import jax.numpy as jnp
from jax.experimental import pallas as pl
from jax.experimental.pallas import tpu as pltpu

ROW_BLOCK = 256


def _colsum_topk_kernel(a_ref, heavy_ref, acc_ref, *, n_row_blocks, k):
    r = pl.program_id(1)
    a = a_ref[0]
    m = jnp.max(a, axis=1, keepdims=True)
    e = jnp.exp(a - m)
    s = jnp.sum(e, axis=1, keepdims=True)
    part = jnp.sum(e / s, axis=0, keepdims=True)

    @pl.when(r == 0)
    def _():
        acc_ref[...] = part

    @pl.when(r > 0)
    def _():
        acc_ref[...] = acc_ref[...] + part

    @pl.when(r == n_row_blocks - 1)
    def _():
        v = acc_ref[...]
        n = v.shape[1]
        vcol = v.reshape(n, 1)
        ii = jax.lax.broadcasted_iota(jnp.int32, (n, n), 0)
        jj = jax.lax.broadcasted_iota(jnp.int32, (n, n), 1)
        beats = (vcol > v) | ((vcol == v) & (ii < jj))
        rank = jnp.sum(beats.astype(jnp.int32), axis=0, keepdims=True)
        heavy_ref[...] = (rank < k).astype(jnp.int32)[None]


def _apply_mask_kernel(a_ref, heavy_ref, o_ref, *, recent, min_value):
    r = pl.program_id(1)
    a = a_ref[0]
    hv = heavy_ref[0]
    rows, cols = a.shape
    i = jax.lax.broadcasted_iota(jnp.int32, (rows, cols), 0) + r * rows
    j = jax.lax.broadcasted_iota(jnp.int32, (rows, cols), 1)
    band = (j <= i + recent) & (j >= i - recent)
    keep = band | (hv != 0)
    o_ref[0] = jnp.where(keep, a, jnp.float32(min_value))


def kernel(attn_weights):
    bs, head, query_len, key_len = attn_weights.shape
    heavy_budget = min(int(0.1 * key_len), key_len)
    recent_budget = int(0.1 * key_len)
    min_value = float(jnp.finfo(attn_weights.dtype).min)

    a = attn_weights.reshape(bs * head, query_len, key_len)
    nh = bs * head
    n_row_blocks = query_len // ROW_BLOCK

    heavy = pl.pallas_call(
        functools.partial(
            _colsum_topk_kernel, n_row_blocks=n_row_blocks, k=heavy_budget
        ),
        grid=(nh, n_row_blocks),
        in_specs=[
            pl.BlockSpec((1, ROW_BLOCK, key_len), lambda h, r: (h, r, 0)),
        ],
        out_specs=pl.BlockSpec((1, 1, key_len), lambda h, r: (h, 0, 0)),
        out_shape=jax.ShapeDtypeStruct((nh, 1, key_len), jnp.int32),
        scratch_shapes=[pltpu.VMEM((1, key_len), jnp.float32)],
    )(a)

    out = pl.pallas_call(
        functools.partial(
            _apply_mask_kernel, recent=recent_budget, min_value=min_value
        ),
        grid=(nh, n_row_blocks),
        in_specs=[
            pl.BlockSpec((1, ROW_BLOCK, key_len), lambda h, r: (h, r, 0)),
            pl.BlockSpec((1, 1, key_len), lambda h, r: (h, 0, 0)),
        ],
        out_specs=pl.BlockSpec((1, ROW_BLOCK, key_len), lambda h, r: (h, r, 0)),
        out_shape=jax.ShapeDtypeStruct((nh, query_len, key_len), jnp.float32),
    )(a, heavy)

    return out.reshape(bs, head, query_len, key_len)
